# baseline (speedup 1.0000x reference)
import functools

import jax
import jax.numpy as jnp
from jax import lax
from jax.experimental import pallas as pl
from jax.experimental.pallas import tpu as pltpu
from jax.experimental.pallas import tpu_sc as plsc

_N = 10000
_NP = 10240
_D = 128
_G = 128
_E = 320000
_NT = 16
_NC = 2
_CH = 128
_SEC = 32
_SECS = ((0, 32), (32, 32), (64, 32), (96, 32), (128, 29))
_CPT = 157
_EPT = _CPT * _CH
_EPB = _EPT * _NT
_RPT = _NP // _NT

_f32 = jnp.float32


def _sc_degree(dst_idx, zeros_np):
    mesh = plsc.VectorSubcoreMesh(core_axis_name="c", subcore_axis_name="s")

    @functools.partial(
        pl.kernel,
        mesh=mesh,
        out_type=jax.ShapeDtypeStruct((_NC, _NP), _f32),
        scratch_types=[
            pltpu.VMEM((_CPT, _CH), jnp.int32),
            pltpu.VMEM((_CH,), _f32),
            pltpu.VMEM_SHARED((_NP,), _f32),
        ],
    )
    def k(dst_hbm, zero_hbm, deg_hbm, idx_v, ones_v, hist_sh):
        c = lax.axis_index("c")
        s = lax.axis_index("s")
        pltpu.sync_copy(dst_hbm.at[c, s], idx_v)
        pltpu.sync_copy(zero_hbm.at[pl.ds(s * _RPT, _RPT)],
                        hist_sh.at[pl.ds(s * _RPT, _RPT)])
        for i in range(_CH // 16):
            ones_v[pl.ds(i * 16, 16)] = jnp.ones((16,), _f32)
        plsc.subcore_barrier()

        def body(j, carry):
            pltpu.sync_copy(ones_v, hist_sh.at[idx_v.at[j]], add=True)
            return carry

        lax.fori_loop(0, _CPT, body, 0)
        plsc.subcore_barrier()
        pltpu.sync_copy(hist_sh.at[pl.ds(s * _RPT, _RPT)],
                        deg_hbm.at[c, pl.ds(s * _RPT, _RPT)])

    return k(dst_idx, zeros_np)


def _sc_scatter(xs, src_idx, dst_idx):
    mesh = plsc.VectorSubcoreMesh(core_axis_name="c", subcore_axis_name="s")

    @functools.partial(
        pl.kernel,
        mesh=mesh,
        out_type=jax.ShapeDtypeStruct((_NC, _NP, _D), _f32),
        scratch_types=[
            pltpu.VMEM((_SEC, _CH), jnp.int32),
            pltpu.VMEM((_SEC, _CH), jnp.int32),
            pltpu.VMEM((_CH, _D), _f32),
            pltpu.VMEM((_CH, _D), _f32),
            pltpu.VMEM_SHARED((_NP, _D), _f32),
            pltpu.SemaphoreType.DMA,
            pltpu.SemaphoreType.DMA,
        ],
    )
    def k(xs_hbm, src_hbm, dst_hbm, acc_hbm,
          srcv, dstv, r0, r1, acc_sh, sem0, sem1):
        c = lax.axis_index("c")
        s = lax.axis_index("s")
        pltpu.sync_copy(xs_hbm.at[c, pl.ds(s * _RPT, _RPT)],
                        acc_sh.at[pl.ds(s * _RPT, _RPT)])
        plsc.subcore_barrier()

        for off, sz in _SECS:
            pltpu.sync_copy(src_hbm.at[c, s, pl.ds(off, sz)],
                            srcv.at[pl.ds(0, sz)])
            pltpu.sync_copy(dst_hbm.at[c, s, pl.ds(off, sz)],
                            dstv.at[pl.ds(0, sz)])
            pltpu.async_copy(xs_hbm.at[c].at[srcv.at[0]], r0, sem0)

            def body(kk, carry):
                j0 = 2 * kk
                j1 = j0 + 1
                pltpu.async_copy(xs_hbm.at[c].at[srcv.at[j1]], r1, sem1)
                pltpu.make_async_copy(xs_hbm.at[c].at[srcv.at[j0]], r0, sem0).wait()
                pltpu.sync_copy(r0, acc_sh.at[dstv.at[j0]], add=True)
                pltpu.async_copy(xs_hbm.at[c].at[srcv.at[j0 + 2]], r0, sem0)
                pltpu.make_async_copy(xs_hbm.at[c].at[srcv.at[j1]], r1, sem1).wait()
                pltpu.sync_copy(r1, acc_sh.at[dstv.at[j1]], add=True)
                return carry

            lax.fori_loop(0, (sz - 1) // 2, body, 0)
            if sz % 2 == 0:
                pltpu.async_copy(xs_hbm.at[c].at[srcv.at[sz - 1]], r1, sem1)
                pltpu.make_async_copy(xs_hbm.at[c].at[srcv.at[sz - 2]],
                                      r0, sem0).wait()
                pltpu.sync_copy(r0, acc_sh.at[dstv.at[sz - 2]], add=True)
                pltpu.make_async_copy(xs_hbm.at[c].at[srcv.at[sz - 1]],
                                      r1, sem1).wait()
                pltpu.sync_copy(r1, acc_sh.at[dstv.at[sz - 1]], add=True)
            else:
                pltpu.make_async_copy(xs_hbm.at[c].at[srcv.at[sz - 1]],
                                      r0, sem0).wait()
                pltpu.sync_copy(r0, acc_sh.at[dstv.at[sz - 1]], add=True)

        plsc.subcore_barrier()
        pltpu.sync_copy(acc_sh.at[pl.ds(s * _RPT, _RPT)],
                        acc_hbm.at[c, pl.ds(s * _RPT, _RPT)])

    return k(xs, src_idx, dst_idx)


_BM = 2048


def _tc_xw1(x01, W):
    M = 2 * _NP

    def body(x_ref, w_ref, o_ref):
        o_ref[...] = jnp.dot(x_ref[...], w_ref[...],
                             preferred_element_type=_f32)

    return pl.pallas_call(
        body,
        grid=(M // _BM,),
        in_specs=[
            pl.BlockSpec((_BM, _D), lambda i: (i, 0)),
            pl.BlockSpec((_D, _D), lambda i: (0, 0)),
        ],
        out_specs=pl.BlockSpec((_BM, _D), lambda i: (i, 0)),
        out_shape=jax.ShapeDtypeStruct((M, _D), _f32),
    )(x01, W)


def _tc_scale(xw, deg):
    M = 2 * _NP

    def body(x_ref, d_ref, o_ref):
        o_ref[...] = x_ref[...] * lax.rsqrt(d_ref[...] + 1.0)

    return pl.pallas_call(
        body,
        grid=(M // _BM,),
        in_specs=[
            pl.BlockSpec((_BM, _D), lambda i: (i, 0)),
            pl.BlockSpec((_BM, 1), lambda i: (i, 0)),
        ],
        out_specs=pl.BlockSpec((_BM, _D), lambda i: (i, 0)),
        out_shape=jax.ShapeDtypeStruct((M, _D), _f32),
    )(xw, deg)


def _tc_xs2(acc, deg, b, W):
    M = 2 * _NP

    def body(a_ref, d_ref, b_ref, w_ref, o_ref):
        dinv = lax.rsqrt(d_ref[...] + 1.0)
        h = jnp.maximum(a_ref[...] * dinv + b_ref[...], 0.0)
        o_ref[...] = jnp.dot(h, w_ref[...],
                             preferred_element_type=_f32) * dinv

    return pl.pallas_call(
        body,
        grid=(M // _BM,),
        in_specs=[
            pl.BlockSpec((_BM, _D), lambda i: (i, 0)),
            pl.BlockSpec((_BM, 1), lambda i: (i, 0)),
            pl.BlockSpec((1, _D), lambda i: (0, 0)),
            pl.BlockSpec((_D, _D), lambda i: (0, 0)),
        ],
        out_specs=pl.BlockSpec((_BM, _D), lambda i: (i, 0)),
        out_shape=jax.ShapeDtypeStruct((M, _D), _f32),
    )(acc, deg, b, W)


def _tc_poolhead(acc2, deg, cb, batchp,
                 w1, b1, w2, b2, g, bta, fcWp, fcbp, ycol, yrow):
    BM = 1024
    M = 2 * _NP
    spb = _NP // BM
    last = M // BM - 1

    def body(a_ref, d_ref, cb_ref, bt_ref,
             w1_ref, b1_ref, w2_ref, b2_ref, g_ref, bt2_ref,
             fw_ref, fb_ref, yc_ref, yr_ref,
             out_ref, loss_ref, p_ref, c_ref):
        i = pl.program_id(0)
        dinv = lax.rsqrt(d_ref[...] + 1.0)
        x2 = jnp.maximum(a_ref[...] * dinv + cb_ref[...], 0.0)
        gids = lax.broadcasted_iota(jnp.int32, (1, _G), 1).astype(_f32)
        m = (bt_ref[...] == gids).astype(_f32)
        pm = lax.dot_general(m, x2, (((0,), (0,)), ((), ())),
                             preferred_element_type=_f32)

        @pl.when(i == 0)
        def _():
            p_ref[0] = pm

        @pl.when((i > 0) & (i < spb))
        def _():
            p_ref[0] += pm

        @pl.when(i == spb)
        def _():
            p_ref[1] = pm

        @pl.when(i > spb)
        def _():
            p_ref[1] += pm

        ones = jnp.ones((BM, _D), _f32)
        cm = lax.dot_general(m, ones, (((0,), (0,)), ((), ())),
                             preferred_element_type=_f32)

        @pl.when(i == 0)
        def _():
            c_ref[...] = cm

        @pl.when((i != 0) & (i < spb))
        def _():
            c_ref[...] += cm

        @pl.when(i == last)
        def _():
            _head(p_ref, c_ref, w1_ref, b1_ref, w2_ref, b2_ref, g_ref,
                  bt2_ref, fw_ref, fb_ref, yc_ref, yr_ref, out_ref, loss_ref)

    def _head(p_ref, c_ref, w1_ref, b1_ref, w2_ref, b2_ref, g_ref, bt_ref,
              fw_ref, fb_ref, yc_ref, yr_ref, out_ref, loss_ref):
        cnt = jnp.maximum(c_ref[...], 1.0)
        x1m = p_ref[0] / cnt
        x2m = p_ref[1] / cnt

        def hfc(xx):
            h = jnp.maximum(jnp.dot(xx, w1_ref[...],
                                    preferred_element_type=_f32)
                            + b1_ref[...], 0.0)
            h = jnp.dot(h, w2_ref[...],
                        preferred_element_type=_f32) + b2_ref[...] + xx
            mu = jnp.mean(h, axis=-1, keepdims=True)
            var = jnp.mean((h - mu) * (h - mu), axis=-1, keepdims=True)
            return g_ref[...] * (h - mu) * lax.rsqrt(var + 1e-6) + bt_ref[...]

        xc = jnp.concatenate(
            [jnp.concatenate([x1m, hfc(x1m)], axis=1),
             jnp.concatenate([x2m, hfc(x2m)], axis=1)], axis=0)

        dot = lax.dot_general(xc, xc, (((1,), (1,)), ((), ())),
                              preferred_element_type=_f32)
        xn = jnp.sqrt(jnp.sum(xc * xc, axis=1, keepdims=True))
        nm = lax.dot_general(xn, xn, (((1,), (1,)), ((), ())),
                             preferred_element_type=_f32)
        cos = jnp.exp(dot / nm / 0.3)
        ri = lax.broadcasted_iota(jnp.int32, (2 * _G, 2 * _G), 0)
        ci = lax.broadcasted_iota(jnp.int32, (2 * _G, 2 * _G), 1)
        cos = jnp.where(ri == ci, 0.0, cos)
        ymat = (yc_ref[...] != yr_ref[...]).astype(_f32)
        neg = cos * ymat
        pos = (cos * (1.0 - ymat))[:_G]
        negsum = jnp.sum(neg, axis=1, keepdims=True)[:_G]
        div = jnp.sum(pos / negsum, axis=1, keepdims=True) / 128.0
        closs = -jnp.sum(jnp.log(div))

        logits = jnp.dot(xc, fw_ref[...],
                         preferred_element_type=_f32) + fb_ref[...]
        valid = lax.broadcasted_iota(jnp.int32, (2 * _G, 128), 1) < 4
        z = jnp.where(valid, logits, -jnp.inf)
        zm = jnp.max(z, axis=1, keepdims=True)
        lse = jnp.log(jnp.sum(jnp.where(valid, jnp.exp(z - zm), 0.0),
                              axis=1, keepdims=True)) + zm
        out_ref[...] = z - lse
        loss_ref[...] = jnp.full((8, 128), closs, _f32)

    cm = lambda i: (0, 0)
    return pl.pallas_call(
        body,
        grid=(M // BM,),
        in_specs=[
            pl.BlockSpec((BM, _D), lambda i: (i, 0)),
            pl.BlockSpec((BM, 1), lambda i: (i, 0)),
            pl.BlockSpec((1, _D), cm),
            pl.BlockSpec((BM, 1), lambda i: (i, 0)),
            pl.BlockSpec((_D, _D), cm),
            pl.BlockSpec((1, _D), cm),
            pl.BlockSpec((_D, _D), cm),
            pl.BlockSpec((1, _D), cm),
            pl.BlockSpec((1, _D), cm),
            pl.BlockSpec((1, _D), cm),
            pl.BlockSpec((2 * _D, 128), cm),
            pl.BlockSpec((1, 128), cm),
            pl.BlockSpec((2 * _G, 1), cm),
            pl.BlockSpec((1, 2 * _G), cm),
        ],
        out_specs=[
            pl.BlockSpec((2 * _G, 128), cm),
            pl.BlockSpec((8, 128), cm),
        ],
        out_shape=[
            jax.ShapeDtypeStruct((2 * _G, 128), _f32),
            jax.ShapeDtypeStruct((8, 128), _f32),
        ],
        scratch_shapes=[
            pltpu.VMEM((2, _G, _D), _f32),
            pltpu.VMEM((_G, _D), _f32),
        ],
    )(acc2, deg, cb, batchp, w1, b1, w2, b2, g, bta, fcWp, fcbp, ycol, yrow)


def kernel(x0, x, edge_index, edge_index2, batch, y1, y2,
           conv1_W, conv1_b, conv2_W, conv2_b,
           h_w1_W, h_w1_b, h_w2_W, h_w2_b, ln_g, ln_b, fc_W, fc_b):
    xp = jnp.zeros((2, _NP, _D), _f32)
    xp = xp.at[0, :_N].set(x0).at[1, :_N].set(x)
    x01 = xp.reshape(2 * _NP, _D)

    pad = _EPB - _E
    padi = jnp.arange(pad, dtype=jnp.int32)
    trash = _N + (padi % 128)

    def prep(v):
        return jnp.concatenate([v, trash]).reshape(_NT, _CPT, _CH)

    dsts = jnp.stack([prep(edge_index[1]), prep(edge_index2[1])])
    degc = _sc_degree(dsts, jnp.zeros((_NP,), _f32))
    xw1 = _tc_xw1(x01, conv1_W)
    srcs = jnp.stack([prep(edge_index[0]), prep(edge_index2[0])])
    deg = degc.reshape(2 * _NP, 1)

    xs1 = _tc_scale(xw1, deg)
    acc1 = _sc_scatter(xs1.reshape(2, _NP, _D), srcs, dsts)
    xs2 = _tc_xs2(acc1.reshape(2 * _NP, _D), deg,
                  conv1_b.reshape(1, _D), conv2_W)
    acc2 = _sc_scatter(xs2.reshape(2, _NP, _D), srcs, dsts)

    bf = batch.astype(_f32)
    batchp = jnp.full((2, _NP), float(_G), _f32)
    batchp = batchp.at[0, :_N].set(bf).at[1, :_N].set(bf)

    fcWp = jnp.zeros((2 * _D, 128), _f32).at[:, :4].set(fc_W)
    fcbp = jnp.zeros((1, 128), _f32).at[0, :4].set(fc_b)
    y = jnp.concatenate([y1, y2])
    yf = y.astype(_f32)
    outp, lossb = _tc_poolhead(acc2.reshape(2 * _NP, _D), deg,
                               conv2_b.reshape(1, _D),
                               batchp.reshape(2 * _NP, 1),
                               h_w1_W, h_w1_b.reshape(1, _D),
                               h_w2_W, h_w2_b.reshape(1, _D),
                               ln_g.reshape(1, _D), ln_b.reshape(1, _D),
                               fcWp, fcbp,
                               yf.reshape(2 * _G, 1), yf.reshape(1, 2 * _G))
    return (outp[:, :4], lossb[0, 0], y)


# speedup vs baseline: 1.0046x; 1.0046x over previous
import functools

import jax
import jax.numpy as jnp
from jax import lax
from jax.experimental import pallas as pl
from jax.experimental.pallas import tpu as pltpu
from jax.experimental.pallas import tpu_sc as plsc

_N = 10000
_NP = 10240
_D = 128
_G = 128
_E = 320000
_NT = 16
_NC = 2
_CH = 128
_SEC = 32
_SECS = ((0, 32), (32, 32), (64, 32), (96, 32), (128, 29))
_CPT = 157
_EPT = _CPT * _CH
_EPB = _EPT * _NT
_RPT = _NP // _NT

_f32 = jnp.float32


def _sc_degree(dst_idx, zeros_np):
    mesh = plsc.VectorSubcoreMesh(core_axis_name="c", subcore_axis_name="s")

    @functools.partial(
        pl.kernel,
        mesh=mesh,
        out_type=jax.ShapeDtypeStruct((_NC, _NP), _f32),
        scratch_types=[
            pltpu.VMEM((_CPT, _CH), jnp.int32),
            pltpu.VMEM((_CH,), _f32),
            pltpu.VMEM_SHARED((_NP,), _f32),
        ],
    )
    def k(dst_hbm, zero_hbm, deg_hbm, idx_v, ones_v, hist_sh):
        c = lax.axis_index("c")
        s = lax.axis_index("s")
        pltpu.sync_copy(dst_hbm.at[c, s], idx_v)
        pltpu.sync_copy(zero_hbm.at[pl.ds(s * _RPT, _RPT)],
                        hist_sh.at[pl.ds(s * _RPT, _RPT)])
        for i in range(_CH // 16):
            ones_v[pl.ds(i * 16, 16)] = jnp.ones((16,), _f32)
        plsc.subcore_barrier()

        def body(j, carry):
            pltpu.sync_copy(ones_v, hist_sh.at[idx_v.at[j]], add=True)
            return carry

        lax.fori_loop(0, _CPT, body, 0)
        plsc.subcore_barrier()
        pltpu.sync_copy(hist_sh.at[pl.ds(s * _RPT, _RPT)],
                        deg_hbm.at[c, pl.ds(s * _RPT, _RPT)])

    return k(dst_idx, zeros_np)


def _sc_scatter(xs, src_idx, dst_idx):
    mesh = plsc.VectorSubcoreMesh(core_axis_name="c", subcore_axis_name="s")

    @functools.partial(
        pl.kernel,
        mesh=mesh,
        out_type=jax.ShapeDtypeStruct((_NC, _NP, _D), _f32),
        scratch_types=[
            pltpu.VMEM((_SEC, _CH), jnp.int32),
            pltpu.VMEM((_SEC, _CH), jnp.int32),
            pltpu.VMEM((_CH, _D), _f32),
            pltpu.VMEM((_CH, _D), _f32),
            pltpu.VMEM_SHARED((_NP, _D), _f32),
            pltpu.SemaphoreType.DMA,
            pltpu.SemaphoreType.DMA,
        ],
    )
    def k(xs_hbm, src_hbm, dst_hbm, acc_hbm,
          srcv, dstv, r0, r1, acc_sh, sem0, sem1):
        c = lax.axis_index("c")
        s = lax.axis_index("s")
        pltpu.sync_copy(xs_hbm.at[c, pl.ds(s * _RPT, _RPT)],
                        acc_sh.at[pl.ds(s * _RPT, _RPT)])
        plsc.subcore_barrier()

        for off, sz in _SECS:
            pltpu.sync_copy(src_hbm.at[c, s, pl.ds(off, sz)],
                            srcv.at[pl.ds(0, sz)])
            pltpu.sync_copy(dst_hbm.at[c, s, pl.ds(off, sz)],
                            dstv.at[pl.ds(0, sz)])
            pltpu.async_copy(xs_hbm.at[c].at[srcv.at[0]], r0, sem0)

            def body(kk, carry):
                j0 = 2 * kk
                j1 = j0 + 1
                pltpu.async_copy(xs_hbm.at[c].at[srcv.at[j1]], r1, sem1)
                pltpu.make_async_copy(xs_hbm.at[c].at[srcv.at[j0]], r0, sem0).wait()
                pltpu.sync_copy(r0, acc_sh.at[dstv.at[j0]], add=True)
                pltpu.async_copy(xs_hbm.at[c].at[srcv.at[j0 + 2]], r0, sem0)
                pltpu.make_async_copy(xs_hbm.at[c].at[srcv.at[j1]], r1, sem1).wait()
                pltpu.sync_copy(r1, acc_sh.at[dstv.at[j1]], add=True)
                return carry

            lax.fori_loop(0, (sz - 1) // 2, body, 0)
            if sz % 2 == 0:
                pltpu.async_copy(xs_hbm.at[c].at[srcv.at[sz - 1]], r1, sem1)
                pltpu.make_async_copy(xs_hbm.at[c].at[srcv.at[sz - 2]],
                                      r0, sem0).wait()
                pltpu.sync_copy(r0, acc_sh.at[dstv.at[sz - 2]], add=True)
                pltpu.make_async_copy(xs_hbm.at[c].at[srcv.at[sz - 1]],
                                      r1, sem1).wait()
                pltpu.sync_copy(r1, acc_sh.at[dstv.at[sz - 1]], add=True)
            else:
                pltpu.make_async_copy(xs_hbm.at[c].at[srcv.at[sz - 1]],
                                      r0, sem0).wait()
                pltpu.sync_copy(r0, acc_sh.at[dstv.at[sz - 1]], add=True)

        plsc.subcore_barrier()
        pltpu.sync_copy(acc_sh.at[pl.ds(s * _RPT, _RPT)],
                        acc_hbm.at[c, pl.ds(s * _RPT, _RPT)])

    return k(xs, src_idx, dst_idx)


_BM = 2048


def _tc_xs1(x01, W, deg):
    M = 2 * _NP

    def body(x_ref, w_ref, d_ref, o_ref):
        dinv = lax.rsqrt(d_ref[...] + 1.0)
        o_ref[...] = jnp.dot(x_ref[...], w_ref[...],
                             preferred_element_type=_f32) * dinv

    return pl.pallas_call(
        body,
        grid=(M // _BM,),
        in_specs=[
            pl.BlockSpec((_BM, _D), lambda i: (i, 0)),
            pl.BlockSpec((_D, _D), lambda i: (0, 0)),
            pl.BlockSpec((_BM, 1), lambda i: (i, 0)),
        ],
        out_specs=pl.BlockSpec((_BM, _D), lambda i: (i, 0)),
        out_shape=jax.ShapeDtypeStruct((M, _D), _f32),
    )(x01, W, deg)


def _tc_xs2(acc, deg, b, W):
    M = 2 * _NP

    def body(a_ref, d_ref, b_ref, w_ref, o_ref):
        dinv = lax.rsqrt(d_ref[...] + 1.0)
        h = jnp.maximum(a_ref[...] * dinv + b_ref[...], 0.0)
        o_ref[...] = jnp.dot(h, w_ref[...],
                             preferred_element_type=_f32) * dinv

    return pl.pallas_call(
        body,
        grid=(M // _BM,),
        in_specs=[
            pl.BlockSpec((_BM, _D), lambda i: (i, 0)),
            pl.BlockSpec((_BM, 1), lambda i: (i, 0)),
            pl.BlockSpec((1, _D), lambda i: (0, 0)),
            pl.BlockSpec((_D, _D), lambda i: (0, 0)),
        ],
        out_specs=pl.BlockSpec((_BM, _D), lambda i: (i, 0)),
        out_shape=jax.ShapeDtypeStruct((M, _D), _f32),
    )(acc, deg, b, W)


def _tc_poolhead(acc2, deg, cb, batchp,
                 w1, b1, w2, b2, g, bta, fcWp, fcbp, ycol, yrow):
    BM = 1024
    M = 2 * _NP
    spb = _NP // BM
    last = M // BM - 1

    def body(a_ref, d_ref, cb_ref, bt_ref,
             w1_ref, b1_ref, w2_ref, b2_ref, g_ref, bt2_ref,
             fw_ref, fb_ref, yc_ref, yr_ref,
             out_ref, loss_ref, p_ref, c_ref):
        i = pl.program_id(0)
        dinv = lax.rsqrt(d_ref[...] + 1.0)
        x2 = jnp.maximum(a_ref[...] * dinv + cb_ref[...], 0.0)
        gids = lax.broadcasted_iota(jnp.int32, (1, _G), 1).astype(_f32)
        m = (bt_ref[...] == gids).astype(_f32)
        pm = lax.dot_general(m, x2, (((0,), (0,)), ((), ())),
                             preferred_element_type=_f32)

        @pl.when(i == 0)
        def _():
            p_ref[0] = pm

        @pl.when((i > 0) & (i < spb))
        def _():
            p_ref[0] += pm

        @pl.when(i == spb)
        def _():
            p_ref[1] = pm

        @pl.when(i > spb)
        def _():
            p_ref[1] += pm

        ones = jnp.ones((BM, _D), _f32)
        cm = lax.dot_general(m, ones, (((0,), (0,)), ((), ())),
                             preferred_element_type=_f32)

        @pl.when(i == 0)
        def _():
            c_ref[...] = cm

        @pl.when((i != 0) & (i < spb))
        def _():
            c_ref[...] += cm

        @pl.when(i == last)
        def _():
            _head(p_ref, c_ref, w1_ref, b1_ref, w2_ref, b2_ref, g_ref,
                  bt2_ref, fw_ref, fb_ref, yc_ref, yr_ref, out_ref, loss_ref)

    def _head(p_ref, c_ref, w1_ref, b1_ref, w2_ref, b2_ref, g_ref, bt_ref,
              fw_ref, fb_ref, yc_ref, yr_ref, out_ref, loss_ref):
        cnt = jnp.maximum(c_ref[...], 1.0)
        x1m = p_ref[0] / cnt
        x2m = p_ref[1] / cnt

        def hfc(xx):
            h = jnp.maximum(jnp.dot(xx, w1_ref[...],
                                    preferred_element_type=_f32)
                            + b1_ref[...], 0.0)
            h = jnp.dot(h, w2_ref[...],
                        preferred_element_type=_f32) + b2_ref[...] + xx
            mu = jnp.mean(h, axis=-1, keepdims=True)
            var = jnp.mean((h - mu) * (h - mu), axis=-1, keepdims=True)
            return g_ref[...] * (h - mu) * lax.rsqrt(var + 1e-6) + bt_ref[...]

        xc = jnp.concatenate(
            [jnp.concatenate([x1m, hfc(x1m)], axis=1),
             jnp.concatenate([x2m, hfc(x2m)], axis=1)], axis=0)

        dot = lax.dot_general(xc, xc, (((1,), (1,)), ((), ())),
                              preferred_element_type=_f32)
        xn = jnp.sqrt(jnp.sum(xc * xc, axis=1, keepdims=True))
        nm = lax.dot_general(xn, xn, (((1,), (1,)), ((), ())),
                             preferred_element_type=_f32)
        cos = jnp.exp(dot / nm / 0.3)
        ri = lax.broadcasted_iota(jnp.int32, (2 * _G, 2 * _G), 0)
        ci = lax.broadcasted_iota(jnp.int32, (2 * _G, 2 * _G), 1)
        cos = jnp.where(ri == ci, 0.0, cos)
        ymat = (yc_ref[...] != yr_ref[...]).astype(_f32)
        neg = cos * ymat
        pos = (cos * (1.0 - ymat))[:_G]
        negsum = jnp.sum(neg, axis=1, keepdims=True)[:_G]
        div = jnp.sum(pos / negsum, axis=1, keepdims=True) / 128.0
        closs = -jnp.sum(jnp.log(div))

        logits = jnp.dot(xc, fw_ref[...],
                         preferred_element_type=_f32) + fb_ref[...]
        valid = lax.broadcasted_iota(jnp.int32, (2 * _G, 128), 1) < 4
        z = jnp.where(valid, logits, -jnp.inf)
        zm = jnp.max(z, axis=1, keepdims=True)
        lse = jnp.log(jnp.sum(jnp.where(valid, jnp.exp(z - zm), 0.0),
                              axis=1, keepdims=True)) + zm
        out_ref[...] = z - lse
        loss_ref[...] = jnp.full((8, 128), closs, _f32)

    cm = lambda i: (0, 0)
    return pl.pallas_call(
        body,
        grid=(M // BM,),
        in_specs=[
            pl.BlockSpec((BM, _D), lambda i: (i, 0)),
            pl.BlockSpec((BM, 1), lambda i: (i, 0)),
            pl.BlockSpec((1, _D), cm),
            pl.BlockSpec((BM, 1), lambda i: (i, 0)),
            pl.BlockSpec((_D, _D), cm),
            pl.BlockSpec((1, _D), cm),
            pl.BlockSpec((_D, _D), cm),
            pl.BlockSpec((1, _D), cm),
            pl.BlockSpec((1, _D), cm),
            pl.BlockSpec((1, _D), cm),
            pl.BlockSpec((2 * _D, 128), cm),
            pl.BlockSpec((1, 128), cm),
            pl.BlockSpec((2 * _G, 1), cm),
            pl.BlockSpec((1, 2 * _G), cm),
        ],
        out_specs=[
            pl.BlockSpec((2 * _G, 128), cm),
            pl.BlockSpec((8, 128), cm),
        ],
        out_shape=[
            jax.ShapeDtypeStruct((2 * _G, 128), _f32),
            jax.ShapeDtypeStruct((8, 128), _f32),
        ],
        scratch_shapes=[
            pltpu.VMEM((2, _G, _D), _f32),
            pltpu.VMEM((_G, _D), _f32),
        ],
    )(acc2, deg, cb, batchp, w1, b1, w2, b2, g, bta, fcWp, fcbp, ycol, yrow)


def kernel(x0, x, edge_index, edge_index2, batch, y1, y2,
           conv1_W, conv1_b, conv2_W, conv2_b,
           h_w1_W, h_w1_b, h_w2_W, h_w2_b, ln_g, ln_b, fc_W, fc_b):
    xp = jnp.zeros((2, _NP, _D), _f32)
    xp = xp.at[0, :_N].set(x0).at[1, :_N].set(x)
    x01 = xp.reshape(2 * _NP, _D)

    pad = _EPB - _E
    padi = jnp.arange(pad, dtype=jnp.int32)
    trash = _N + (padi % 128)

    def prep(v):
        return jnp.concatenate([v, trash]).reshape(_NT, _CPT, _CH)

    dsts = jnp.stack([prep(edge_index[1]), prep(edge_index2[1])])
    degc = _sc_degree(dsts, jnp.zeros((_NP,), _f32))
    srcs = jnp.stack([prep(edge_index[0]), prep(edge_index2[0])])
    deg = degc.reshape(2 * _NP, 1)

    xs1 = _tc_xs1(x01, conv1_W, deg)
    acc1 = _sc_scatter(xs1.reshape(2, _NP, _D), srcs, dsts)
    xs2 = _tc_xs2(acc1.reshape(2 * _NP, _D), deg,
                  conv1_b.reshape(1, _D), conv2_W)
    acc2 = _sc_scatter(xs2.reshape(2, _NP, _D), srcs, dsts)

    bf = batch.astype(_f32)
    batchp = jnp.full((2, _NP), float(_G), _f32)
    batchp = batchp.at[0, :_N].set(bf).at[1, :_N].set(bf)

    fcWp = jnp.zeros((2 * _D, 128), _f32).at[:, :4].set(fc_W)
    fcbp = jnp.zeros((1, 128), _f32).at[0, :4].set(fc_b)
    y = jnp.concatenate([y1, y2])
    yf = y.astype(_f32)
    outp, lossb = _tc_poolhead(acc2.reshape(2 * _NP, _D), deg,
                               conv2_b.reshape(1, _D),
                               batchp.reshape(2 * _NP, 1),
                               h_w1_W, h_w1_b.reshape(1, _D),
                               h_w2_W, h_w2_b.reshape(1, _D),
                               ln_g.reshape(1, _D), ln_b.reshape(1, _D),
                               fcWp, fcbp,
                               yf.reshape(2 * _G, 1), yf.reshape(1, 2 * _G))
    return (outp[:, :4], lossb[0, 0], y)
